# Initial kernel scaffold; baseline (speedup 1.0000x reference)
import jax
import jax.numpy as jnp
from jax.experimental import pallas as pl


def kernel(node_emb, edge_index, relation_emb, W1_src, b1_src, W2_src, b2_src, W1_dst, b1_dst, W2_dst, b2_dst, W1_edge, b1_edge, W2_edge, b2_edge):
    raise NotImplementedError("write your pallas kernel here")


# speedup vs baseline: 9.8571x; 9.8571x over previous
import functools

import numpy as np
import jax
import jax.numpy as jnp
from jax import lax
from jax.experimental import pallas as pl
from jax.experimental.pallas import tpu as pltpu
from jax.experimental.pallas import tpu_sc as plsc

N_NODES = 10000
N_EDGES = 160000
NODE_DIM = 256
EDGE_DIM = 16
HIDDEN = 256

N_PAD = 10240
E_PAD = 163840
NODE_BLK = 2048
EDGE_BLK = 8192
N_WORKERS = 32
CHUNK = E_PAD // N_WORKERS


def _gumbel_const() -> np.ndarray:
    def rotl(x, r):
        return ((x << np.uint32(r)) | (x >> np.uint32(32 - r))).astype(np.uint32)

    ks0, ks1 = np.uint32(0), np.uint32(123)
    ks2 = np.uint32(ks0 ^ ks1 ^ np.uint32(0x1BD11BDA))
    ks = [ks0, ks1, ks2]
    rot = [(13, 15, 26, 6), (17, 29, 16, 24)]
    x0 = np.full(N_EDGES, ks0, np.uint32)
    x1 = (np.arange(N_EDGES, dtype=np.uint32) + ks1).astype(np.uint32)
    for i in range(5):
        for r in rot[i % 2]:
            x0 = (x0 + x1).astype(np.uint32)
            x1 = rotl(x1, r)
            x1 = (x1 ^ x0).astype(np.uint32)
        x0 = (x0 + ks[(i + 1) % 3]).astype(np.uint32)
        x1 = (x1 + ks[(i + 2) % 3] + np.uint32(i + 1)).astype(np.uint32)
    bits = (x0 ^ x1).astype(np.uint32)
    u = ((bits >> np.uint32(9)) | np.uint32(0x3F800000)).view(np.float32) \
        - np.float32(1.0)
    bias = np.float32(0.0001)
    eps = (bias - (np.float32(1.0) - bias)) * u + (np.float32(1.0) - bias)
    g = np.log(eps, dtype=np.float32) - np.log(np.float32(1.0) - eps,
                                               dtype=np.float32)
    out = np.full(E_PAD, -1e9, np.float32)
    out[:N_EDGES] = g
    return out


_GUMBEL = _gumbel_const()


def _node_mlp_body(x_ref, w1s_ref, b1s_ref, w2s_ref,
                   w1d_ref, b1d_ref, w2d_ref, outs_ref, outd_ref):
    x = x_ref[...]
    hs = jnp.maximum(
        jnp.dot(x, w1s_ref[...], preferred_element_type=jnp.float32)
        + b1s_ref[...], 0.0)
    outs_ref[...] = lax.dot_general(
        w2s_ref[...], hs, (((1,), (1,)), ((), ())),
        preferred_element_type=jnp.float32).reshape(NODE_BLK)
    hd = jnp.maximum(
        jnp.dot(x, w1d_ref[...], preferred_element_type=jnp.float32)
        + b1d_ref[...], 0.0)
    outd_ref[...] = lax.dot_general(
        w2d_ref[...], hd, (((1,), (1,)), ((), ())),
        preferred_element_type=jnp.float32).reshape(NODE_BLK)


def _edge_mlp_body(x_ref, w1e_ref, b1e_ref, w2e_ref, b2_ref, gum_ref,
                   out_ref):
    h = jnp.maximum(
        jnp.dot(x_ref[...], w1e_ref[...], preferred_element_type=jnp.float32)
        + b1e_ref[...], 0.0)
    s = lax.dot_general(w2e_ref[...], h, (((1,), (1,)), ((), ())),
                        preferred_element_type=jnp.float32).reshape(EDGE_BLK)
    out_ref[...] = s + b2_ref[0, 0] + gum_ref[...]


def _sc_gate_body(wsrc_hbm, wdst_hbm, src_hbm, dst_hbm, wep_hbm,
                  gate_hbm, part_hbm,
                  tbl_s, tbl_d, idx_s, idx_d, wep_v, gate_v, acc_v):
    wid = lax.axis_index("s") * 2 + lax.axis_index("c")
    base = pl.multiple_of(wid * CHUNK, CHUNK)
    pltpu.sync_copy(wsrc_hbm, tbl_s)
    pltpu.sync_copy(wdst_hbm, tbl_d)
    pltpu.sync_copy(src_hbm.at[pl.ds(base, CHUNK)], idx_s)
    pltpu.sync_copy(dst_hbm.at[pl.ds(base, CHUNK)], idx_d)
    pltpu.sync_copy(wep_hbm.at[pl.ds(base, CHUNK)], wep_v)

    def body(i, acc):
        o = pl.multiple_of(i * 16, 16)
        vs = plsc.load_gather(tbl_s, [idx_s[pl.ds(o, 16)]])
        vd = plsc.load_gather(tbl_d, [idx_d[pl.ds(o, 16)]])
        w = wep_v[pl.ds(o, 16)] + vs + vd
        g = 1.0 / (1.0 + jnp.exp(-2.0 * w))
        gate_v[pl.ds(o, 16)] = g
        return acc + g

    acc = lax.fori_loop(0, CHUNK // 16, body, jnp.zeros((16,), jnp.float32))
    acc_v[...] = acc
    pltpu.sync_copy(gate_v, gate_hbm.at[pl.ds(base, CHUNK)])
    pltpu.sync_copy(acc_v, part_hbm.at[wid])


def kernel(node_emb, edge_index, relation_emb,
           W1_src, b1_src, W2_src, b2_src,
           W1_dst, b1_dst, W2_dst, b2_dst,
           W1_edge, b1_edge, W2_edge, b2_edge):
    f32 = jnp.float32
    src = edge_index[0].astype(jnp.int32)
    dst = edge_index[1].astype(jnp.int32)
    src_p = jnp.pad(src, (0, E_PAD - N_EDGES))
    dst_p = jnp.pad(dst, (0, E_PAD - N_EDGES))
    x_p = jnp.pad(node_emb, ((0, N_PAD - N_NODES), (0, 0)))
    rel_p = jnp.pad(relation_emb, ((0, E_PAD - N_EDGES), (0, 0)))
    b2sum = (b2_src + b2_dst + b2_edge).reshape(1, 1)
    gum = jnp.asarray(_GUMBEL)

    n_grid = N_PAD // NODE_BLK
    full = lambda i: (0, 0)
    node_scores = pl.pallas_call(
        _node_mlp_body,
        grid=(n_grid,),
        in_specs=[
            pl.BlockSpec((NODE_BLK, NODE_DIM), lambda i: (i, 0)),
            pl.BlockSpec((NODE_DIM, HIDDEN), full),
            pl.BlockSpec((1, HIDDEN), full),
            pl.BlockSpec((1, HIDDEN), full),
            pl.BlockSpec((NODE_DIM, HIDDEN), full),
            pl.BlockSpec((1, HIDDEN), full),
            pl.BlockSpec((1, HIDDEN), full),
        ],
        out_specs=[
            pl.BlockSpec((NODE_BLK,), lambda i: (i,)),
            pl.BlockSpec((NODE_BLK,), lambda i: (i,)),
        ],
        out_shape=[
            jax.ShapeDtypeStruct((N_PAD,), f32),
            jax.ShapeDtypeStruct((N_PAD,), f32),
        ],
    )
    wsrc, wdst = node_scores(
        x_p,
        W1_src, b1_src.reshape(1, HIDDEN), W2_src.reshape(1, HIDDEN),
        W1_dst, b1_dst.reshape(1, HIDDEN), W2_dst.reshape(1, HIDDEN),
    )

    e_grid = E_PAD // EDGE_BLK
    wedge_plus = pl.pallas_call(
        _edge_mlp_body,
        grid=(e_grid,),
        in_specs=[
            pl.BlockSpec((EDGE_BLK, EDGE_DIM), lambda i: (i, 0)),
            pl.BlockSpec((EDGE_DIM, HIDDEN), full),
            pl.BlockSpec((1, HIDDEN), full),
            pl.BlockSpec((1, HIDDEN), full),
            pl.BlockSpec((1, 1), full),
            pl.BlockSpec((EDGE_BLK,), lambda i: (i,)),
        ],
        out_specs=pl.BlockSpec((EDGE_BLK,), lambda i: (i,)),
        out_shape=jax.ShapeDtypeStruct((E_PAD,), f32),
    )(rel_p, W1_edge, b1_edge.reshape(1, HIDDEN),
      W2_edge.reshape(1, HIDDEN), b2sum, gum)

    mesh = plsc.VectorSubcoreMesh(core_axis_name="c", subcore_axis_name="s")
    sc_gate = pl.kernel(
        _sc_gate_body,
        out_type=[
            jax.ShapeDtypeStruct((E_PAD,), f32),
            jax.ShapeDtypeStruct((N_WORKERS, 16), f32),
        ],
        mesh=mesh,
        compiler_params=pltpu.CompilerParams(needs_layout_passes=False),
        scratch_types=[
            pltpu.VMEM((N_PAD,), f32),
            pltpu.VMEM((N_PAD,), f32),
            pltpu.VMEM((CHUNK,), jnp.int32),
            pltpu.VMEM((CHUNK,), jnp.int32),
            pltpu.VMEM((CHUNK,), f32),
            pltpu.VMEM((CHUNK,), f32),
            pltpu.VMEM((16,), f32),
        ],
    )
    gate, parts = sc_gate(wsrc, wdst, src_p, dst_p, wedge_plus)

    aug_edge_weight = gate[:N_EDGES].reshape(N_EDGES, 1, 1)
    reg = 1.0 - jnp.sum(parts) / np.float32(N_EDGES)
    return (reg, aug_edge_weight)


# speedup vs baseline: 11.2070x; 1.1370x over previous
import numpy as np
import jax
import jax.numpy as jnp
from jax import lax
from jax.experimental import pallas as pl
from jax.experimental.pallas import tpu as pltpu
from jax.experimental.pallas import tpu_sc as plsc

N_NODES = 10000
N_EDGES = 160000
NODE_DIM = 256
EDGE_DIM = 16
HIDDEN = 256

N_PAD = 10240
NODE_BLK = 2048
EDGE_BLK = 8192
N_WORKERS = 32
E_PAD = 163840
CHUNK = E_PAD // N_WORKERS
LAST_VALID = N_EDGES - (N_WORKERS - 1) * CHUNK


def _gumbel_const() -> np.ndarray:
    def rotl(x, r):
        return ((x << np.uint32(r)) | (x >> np.uint32(32 - r))).astype(np.uint32)

    ks0, ks1 = np.uint32(0), np.uint32(123)
    ks2 = np.uint32(ks0 ^ ks1 ^ np.uint32(0x1BD11BDA))
    ks = [ks0, ks1, ks2]
    rot = [(13, 15, 26, 6), (17, 29, 16, 24)]
    x0 = np.full(N_EDGES, ks0, np.uint32)
    x1 = (np.arange(N_EDGES, dtype=np.uint32) + ks1).astype(np.uint32)
    for i in range(5):
        for r in rot[i % 2]:
            x0 = (x0 + x1).astype(np.uint32)
            x1 = rotl(x1, r)
            x1 = (x1 ^ x0).astype(np.uint32)
        x0 = (x0 + ks[(i + 1) % 3]).astype(np.uint32)
        x1 = (x1 + ks[(i + 2) % 3] + np.uint32(i + 1)).astype(np.uint32)
    bits = (x0 ^ x1).astype(np.uint32)
    u = ((bits >> np.uint32(9)) | np.uint32(0x3F800000)).view(np.float32) \
        - np.float32(1.0)
    bias = np.float32(0.0001)
    eps = (bias - (np.float32(1.0) - bias)) * u + (np.float32(1.0) - bias)
    g = np.log(eps, dtype=np.float32) - np.log(np.float32(1.0) - eps,
               dtype=np.float32)
    out = np.full(E_PAD, -1e9, np.float32)
    out[:N_EDGES] = g
    return out


_GUMBEL = _gumbel_const()


def _node_mlp_body(x_ref, w1s_ref, b1s_ref, w2s_ref,
                   w1d_ref, b1d_ref, w2d_ref, outs_ref, outd_ref):
    x = x_ref[...]
    hs = jnp.maximum(
        jnp.dot(x, w1s_ref[...], preferred_element_type=jnp.float32)
        + b1s_ref[...], 0.0)
    outs_ref[...] = lax.dot_general(
        w2s_ref[...], hs, (((1,), (1,)), ((), ())),
        preferred_element_type=jnp.float32).reshape(NODE_BLK)
    hd = jnp.maximum(
        jnp.dot(x, w1d_ref[...], preferred_element_type=jnp.float32)
        + b1d_ref[...], 0.0)
    outd_ref[...] = lax.dot_general(
        w2d_ref[...], hd, (((1,), (1,)), ((), ())),
        preferred_element_type=jnp.float32).reshape(NODE_BLK)


def _edge_mlp_body(x_ref, w1e_ref, b1e_ref, w2e_ref, b2_ref, gum_ref,
                   out_ref):
    i = pl.program_id(0)
    h = jnp.maximum(
        jnp.dot(x_ref[...], w1e_ref[...], preferred_element_type=jnp.float32)
        + b1e_ref[...], 0.0)
    s = lax.dot_general(w2e_ref[...], h, (((1,), (1,)), ((), ())),
                        preferred_element_type=jnp.float32).reshape(EDGE_BLK)
    valid = i * EDGE_BLK + lax.iota(jnp.int32, EDGE_BLK) < N_EDGES
    out_ref[...] = jnp.where(valid, s + b2_ref[0, 0] + gum_ref[...], -1e9)


def _sc_gate_body(wsrc_hbm, wdst_hbm, src_hbm, dst_hbm, wep_hbm,
                  gate_hbm, part_hbm,
                  tbl_s, tbl_d, idx_s, idx_d, wep_v, gate_v, acc_v):
    wid = lax.axis_index("s") * 2 + lax.axis_index("c")
    base = pl.multiple_of(wid * CHUNK, 128)
    pltpu.sync_copy(wsrc_hbm, tbl_s)
    pltpu.sync_copy(wdst_hbm, tbl_d)
    pltpu.sync_copy(src_hbm.at[pl.ds(base, CHUNK)], idx_s)
    pltpu.sync_copy(dst_hbm.at[pl.ds(base, CHUNK)], idx_d)
    pltpu.sync_copy(wep_hbm.at[pl.ds(base, CHUNK)], wep_v)

    def body(i, acc):
        o = pl.multiple_of(i * 16, 16)
        vs = plsc.load_gather(tbl_s, [idx_s[pl.ds(o, 16)]])
        vd = plsc.load_gather(tbl_d, [idx_d[pl.ds(o, 16)]])
        w = wep_v[pl.ds(o, 16)] + vs + vd
        g = 1.0 / (1.0 + jnp.exp(-2.0 * w))
        gate_v[pl.ds(o, 16)] = g
        return acc + g

    acc = lax.fori_loop(0, CHUNK // 16, body,
                        jnp.zeros((16,), jnp.float32))
    acc_v[...] = acc

    @pl.when(wid < N_WORKERS - 1)
    def _copy_full():
        pltpu.sync_copy(gate_v, gate_hbm.at[pl.ds(base, CHUNK)])

    @pl.when(wid == N_WORKERS - 1)
    def _copy_last():
        pltpu.sync_copy(gate_v.at[pl.ds(0, LAST_VALID)],
                        gate_hbm.at[pl.ds(base, LAST_VALID)])

    pltpu.sync_copy(acc_v, part_hbm.at[wid])


def kernel(node_emb, edge_index, relation_emb,
           W1_src, b1_src, W2_src, b2_src,
           W1_dst, b1_dst, W2_dst, b2_dst,
           W1_edge, b1_edge, W2_edge, b2_edge):
    f32 = jnp.float32
    ei = edge_index.astype(jnp.int32)
    src_p = jnp.pad(ei[0], (0, E_PAD - N_EDGES))
    dst_p = jnp.pad(ei[1], (0, E_PAD - N_EDGES))
    b2sum = (b2_src + b2_dst + b2_edge).reshape(1, 1)
    gum = jnp.asarray(_GUMBEL)

    full = lambda i: (0, 0)
    n_grid = N_PAD // NODE_BLK
    wsrc, wdst = pl.pallas_call(
        _node_mlp_body,
        grid=(n_grid,),
        in_specs=[
            pl.BlockSpec((NODE_BLK, NODE_DIM), lambda i: (i, 0)),
            pl.BlockSpec((NODE_DIM, HIDDEN), full),
            pl.BlockSpec((1, HIDDEN), full),
            pl.BlockSpec((1, HIDDEN), full),
            pl.BlockSpec((NODE_DIM, HIDDEN), full),
            pl.BlockSpec((1, HIDDEN), full),
            pl.BlockSpec((1, HIDDEN), full),
        ],
        out_specs=[
            pl.BlockSpec((NODE_BLK,), lambda i: (i,)),
            pl.BlockSpec((NODE_BLK,), lambda i: (i,)),
        ],
        out_shape=[
            jax.ShapeDtypeStruct((N_PAD,), f32),
            jax.ShapeDtypeStruct((N_PAD,), f32),
        ],
    )(node_emb,
      W1_src, b1_src.reshape(1, HIDDEN), W2_src.reshape(1, HIDDEN),
      W1_dst, b1_dst.reshape(1, HIDDEN), W2_dst.reshape(1, HIDDEN))

    e_grid = E_PAD // EDGE_BLK
    wedge_plus = pl.pallas_call(
        _edge_mlp_body,
        grid=(e_grid,),
        in_specs=[
            pl.BlockSpec((EDGE_BLK, EDGE_DIM), lambda i: (i, 0)),
            pl.BlockSpec((EDGE_DIM, HIDDEN), full),
            pl.BlockSpec((1, HIDDEN), full),
            pl.BlockSpec((1, HIDDEN), full),
            pl.BlockSpec((1, 1), full),
            pl.BlockSpec((EDGE_BLK,), lambda i: (i,)),
        ],
        out_specs=pl.BlockSpec((EDGE_BLK,), lambda i: (i,)),
        out_shape=jax.ShapeDtypeStruct((E_PAD,), f32),
    )(relation_emb, W1_edge, b1_edge.reshape(1, HIDDEN),
      W2_edge.reshape(1, HIDDEN), b2sum, gum)

    mesh = plsc.VectorSubcoreMesh(core_axis_name="c", subcore_axis_name="s")
    sc_gate = pl.kernel(
        _sc_gate_body,
        out_type=[
            jax.ShapeDtypeStruct((N_EDGES,), f32),
            jax.ShapeDtypeStruct((N_WORKERS, 16), f32),
        ],
        mesh=mesh,
        compiler_params=pltpu.CompilerParams(needs_layout_passes=False),
        scratch_types=[
            pltpu.VMEM((N_PAD,), f32),
            pltpu.VMEM((N_PAD,), f32),
            pltpu.VMEM((CHUNK,), jnp.int32),
            pltpu.VMEM((CHUNK,), jnp.int32),
            pltpu.VMEM((CHUNK,), f32),
            pltpu.VMEM((CHUNK,), f32),
            pltpu.VMEM((16,), f32),
        ],
    )
    gate, parts = sc_gate(wsrc, wdst, src_p, dst_p, wedge_plus)

    aug_edge_weight = gate.reshape(N_EDGES, 1, 1)
    reg = 1.0 - jnp.sum(parts) / np.float32(N_EDGES)
    return (reg, aug_edge_weight)


# speedup vs baseline: 11.9277x; 1.0643x over previous
import numpy as np
import jax
import jax.numpy as jnp
from jax import lax
from jax.experimental import pallas as pl
from jax.experimental.pallas import tpu as pltpu
from jax.experimental.pallas import tpu_sc as plsc

N_NODES = 10000
N_EDGES = 160000
NODE_DIM = 256
EDGE_DIM = 16
HIDDEN = 256

GRID = 20
N_PAD = 10240
NODE_BLK = 512
EDGE_BLK = 8192
N_WORKERS = 32
E_PAD = 163840
CHUNK = E_PAD // N_WORKERS
LAST_VALID = N_EDGES - (N_WORKERS - 1) * CHUNK


def _gumbel_const() -> np.ndarray:
    def rotl(x, r):
        return ((x << np.uint32(r)) | (x >> np.uint32(32 - r))).astype(np.uint32)

    ks0, ks1 = np.uint32(0), np.uint32(123)
    ks2 = np.uint32(ks0 ^ ks1 ^ np.uint32(0x1BD11BDA))
    ks = [ks0, ks1, ks2]
    rot = [(13, 15, 26, 6), (17, 29, 16, 24)]
    x0 = np.full(N_EDGES, ks0, np.uint32)
    x1 = (np.arange(N_EDGES, dtype=np.uint32) + ks1).astype(np.uint32)
    for i in range(5):
        for r in rot[i % 2]:
            x0 = (x0 + x1).astype(np.uint32)
            x1 = rotl(x1, r)
            x1 = (x1 ^ x0).astype(np.uint32)
        x0 = (x0 + ks[(i + 1) % 3]).astype(np.uint32)
        x1 = (x1 + ks[(i + 2) % 3] + np.uint32(i + 1)).astype(np.uint32)
    bits = (x0 ^ x1).astype(np.uint32)
    u = ((bits >> np.uint32(9)) | np.uint32(0x3F800000)).view(np.float32) \
        - np.float32(1.0)
    bias = np.float32(0.0001)
    eps = (bias - (np.float32(1.0) - bias)) * u + (np.float32(1.0) - bias)
    g = np.log(eps, dtype=np.float32) - np.log(np.float32(1.0) - eps,
               dtype=np.float32)
    out = np.full(E_PAD, -1e9, np.float32)
    out[:N_EDGES] = g
    return out


_GUMBEL = _gumbel_const()
_BF = jnp.bfloat16


def _score_head(x_bf, w1_ref, b1_ref, w2_ref, n):
    acc = jnp.dot(x_bf, w1_ref[...].astype(_BF),
                  preferred_element_type=jnp.float32)
    h = jnp.maximum(acc.astype(_BF) + b1_ref[...].astype(_BF),
                    jnp.zeros((), _BF))
    return lax.dot_general(
        w2_ref[...].astype(_BF), h, (((1,), (1,)), ((), ())),
        preferred_element_type=jnp.float32).reshape(n)


def _tc_body(x_ref, rel_ref, ei_ref, gum_ref,
             w1s_ref, b1s_ref, w2s_ref,
             w1d_ref, b1d_ref, w2d_ref,
             w1e_ref, b1e_ref, w2e_ref, b2_ref,
             wsrc_ref, wdst_ref, wep_ref, srcp_ref, dstp_ref):
    i = pl.program_id(0)
    x_bf = x_ref[...].astype(_BF)
    wsrc_ref[...] = _score_head(x_bf, w1s_ref, b1s_ref, w2s_ref, NODE_BLK)
    wdst_ref[...] = _score_head(x_bf, w1d_ref, b1d_ref, w2d_ref, NODE_BLK)

    se = _score_head(rel_ref[...].astype(_BF), w1e_ref, b1e_ref, w2e_ref,
                     EDGE_BLK)
    valid = i * EDGE_BLK + lax.iota(jnp.int32, EDGE_BLK) < N_EDGES
    wep_ref[...] = jnp.where(valid, se + b2_ref[0, 0] + gum_ref[...], -1e9)
    zeros = jnp.zeros((EDGE_BLK,), jnp.int32)
    srcp_ref[...] = jnp.where(valid, ei_ref[0, :], zeros)
    dstp_ref[...] = jnp.where(valid, ei_ref[1, :], zeros)


def _sc_gate_body(wsrc_hbm, wdst_hbm, src_hbm, dst_hbm, wep_hbm,
                  gate_hbm, part_hbm,
                  tbl_s, tbl_d, idx_s, idx_d, wep_v, gate_v, acc_v):
    wid = lax.axis_index("s") * 2 + lax.axis_index("c")
    base = pl.multiple_of(wid * CHUNK, 128)
    pltpu.sync_copy(wsrc_hbm, tbl_s)
    pltpu.sync_copy(wdst_hbm, tbl_d)
    pltpu.sync_copy(src_hbm.at[pl.ds(base, CHUNK)], idx_s)
    pltpu.sync_copy(dst_hbm.at[pl.ds(base, CHUNK)], idx_d)
    pltpu.sync_copy(wep_hbm.at[pl.ds(base, CHUNK)], wep_v)

    def body(i, acc):
        o = pl.multiple_of(i * 16, 16)
        vs = plsc.load_gather(tbl_s, [idx_s[pl.ds(o, 16)]])
        vd = plsc.load_gather(tbl_d, [idx_d[pl.ds(o, 16)]])
        w = wep_v[pl.ds(o, 16)] + vs + vd
        g = 1.0 / (1.0 + jnp.exp(-2.0 * w))
        gate_v[pl.ds(o, 16)] = g
        return acc + g

    acc = lax.fori_loop(0, CHUNK // 16, body,
                        jnp.zeros((16,), jnp.float32))
    acc_v[...] = acc

    @pl.when(wid < N_WORKERS - 1)
    def _copy_full():
        pltpu.sync_copy(gate_v, gate_hbm.at[pl.ds(base, CHUNK)])

    @pl.when(wid == N_WORKERS - 1)
    def _copy_last():
        pltpu.sync_copy(gate_v.at[pl.ds(0, LAST_VALID)],
                        gate_hbm.at[pl.ds(base, LAST_VALID)])

    pltpu.sync_copy(acc_v, part_hbm.at[wid])


def kernel(node_emb, edge_index, relation_emb,
           W1_src, b1_src, W2_src, b2_src,
           W1_dst, b1_dst, W2_dst, b2_dst,
           W1_edge, b1_edge, W2_edge, b2_edge):
    f32 = jnp.float32
    ei = edge_index.astype(jnp.int32)
    b2sum = (b2_src + b2_dst + b2_edge).reshape(1, 1)
    gum = jnp.asarray(_GUMBEL)

    full = lambda i: (0, 0)
    wsrc, wdst, wep, src_p, dst_p = pl.pallas_call(
        _tc_body,
        grid=(GRID,),
        in_specs=[
            pl.BlockSpec((NODE_BLK, NODE_DIM), lambda i: (i, 0)),
            pl.BlockSpec((EDGE_BLK, EDGE_DIM), lambda i: (i, 0)),
            pl.BlockSpec((2, EDGE_BLK), lambda i: (0, i)),
            pl.BlockSpec((EDGE_BLK,), lambda i: (i,)),
            pl.BlockSpec((NODE_DIM, HIDDEN), full),
            pl.BlockSpec((1, HIDDEN), full),
            pl.BlockSpec((1, HIDDEN), full),
            pl.BlockSpec((NODE_DIM, HIDDEN), full),
            pl.BlockSpec((1, HIDDEN), full),
            pl.BlockSpec((1, HIDDEN), full),
            pl.BlockSpec((EDGE_DIM, HIDDEN), full),
            pl.BlockSpec((1, HIDDEN), full),
            pl.BlockSpec((1, HIDDEN), full),
            pl.BlockSpec((1, 1), full),
        ],
        out_specs=[
            pl.BlockSpec((NODE_BLK,), lambda i: (i,)),
            pl.BlockSpec((NODE_BLK,), lambda i: (i,)),
            pl.BlockSpec((EDGE_BLK,), lambda i: (i,)),
            pl.BlockSpec((EDGE_BLK,), lambda i: (i,)),
            pl.BlockSpec((EDGE_BLK,), lambda i: (i,)),
        ],
        out_shape=[
            jax.ShapeDtypeStruct((N_PAD,), f32),
            jax.ShapeDtypeStruct((N_PAD,), f32),
            jax.ShapeDtypeStruct((E_PAD,), f32),
            jax.ShapeDtypeStruct((E_PAD,), jnp.int32),
            jax.ShapeDtypeStruct((E_PAD,), jnp.int32),
        ],
    )(node_emb, relation_emb, ei, gum,
      W1_src, b1_src.reshape(1, HIDDEN), W2_src.reshape(1, HIDDEN),
      W1_dst, b1_dst.reshape(1, HIDDEN), W2_dst.reshape(1, HIDDEN),
      W1_edge, b1_edge.reshape(1, HIDDEN), W2_edge.reshape(1, HIDDEN),
      b2sum)

    mesh = plsc.VectorSubcoreMesh(core_axis_name="c", subcore_axis_name="s")
    sc_gate = pl.kernel(
        _sc_gate_body,
        out_type=[
            jax.ShapeDtypeStruct((N_EDGES,), f32),
            jax.ShapeDtypeStruct((N_WORKERS, 16), f32),
        ],
        mesh=mesh,
        compiler_params=pltpu.CompilerParams(needs_layout_passes=False),
        scratch_types=[
            pltpu.VMEM((N_PAD,), f32),
            pltpu.VMEM((N_PAD,), f32),
            pltpu.VMEM((CHUNK,), jnp.int32),
            pltpu.VMEM((CHUNK,), jnp.int32),
            pltpu.VMEM((CHUNK,), f32),
            pltpu.VMEM((CHUNK,), f32),
            pltpu.VMEM((16,), f32),
        ],
    )
    gate, parts = sc_gate(wsrc, wdst, src_p, dst_p, wep)

    aug_edge_weight = gate.reshape(N_EDGES, 1, 1)
    reg = 1.0 - jnp.sum(parts) / np.float32(N_EDGES)
    return (reg, aug_edge_weight)


# speedup vs baseline: 18.2524x; 1.5303x over previous
import numpy as np
import jax
import jax.numpy as jnp
from jax import lax
from jax.experimental import pallas as pl
from jax.experimental.pallas import tpu as pltpu
from jax.experimental.pallas import tpu_sc as plsc

N_NODES = 10000
N_EDGES = 160000
NODE_DIM = 256
EDGE_DIM = 16
HIDDEN = 256

GRID = 10
N_PAD = 10240
NODE_BLK = 1024
EDGE_BLK = 16384
N_WORKERS = 32
E_PAD = 163840
CHUNK = E_PAD // N_WORKERS


def _gumbel_const() -> np.ndarray:
    def rotl(x, r):
        return ((x << np.uint32(r)) | (x >> np.uint32(32 - r))).astype(np.uint32)

    ks0, ks1 = np.uint32(0), np.uint32(123)
    ks2 = np.uint32(ks0 ^ ks1 ^ np.uint32(0x1BD11BDA))
    ks = [ks0, ks1, ks2]
    rot = [(13, 15, 26, 6), (17, 29, 16, 24)]
    x0 = np.full(N_EDGES, ks0, np.uint32)
    x1 = (np.arange(N_EDGES, dtype=np.uint32) + ks1).astype(np.uint32)
    for i in range(5):
        for r in rot[i % 2]:
            x0 = (x0 + x1).astype(np.uint32)
            x1 = rotl(x1, r)
            x1 = (x1 ^ x0).astype(np.uint32)
        x0 = (x0 + ks[(i + 1) % 3]).astype(np.uint32)
        x1 = (x1 + ks[(i + 2) % 3] + np.uint32(i + 1)).astype(np.uint32)
    bits = (x0 ^ x1).astype(np.uint32)
    u = ((bits >> np.uint32(9)) | np.uint32(0x3F800000)).view(np.float32) \
        - np.float32(1.0)
    bias = np.float32(0.0001)
    eps = (bias - (np.float32(1.0) - bias)) * u + (np.float32(1.0) - bias)
    g = np.log(eps, dtype=np.float32) - np.log(np.float32(1.0) - eps,
               dtype=np.float32)
    out = np.full(E_PAD, -1e9, np.float32)
    out[:N_EDGES] = g
    return out


_GUMBEL = _gumbel_const()
_BF = jnp.bfloat16


def _score_head(x_bf, w1_ref, b1_ref, w2_ref, n):
    acc = jnp.dot(x_bf, w1_ref[...].astype(_BF),
                  preferred_element_type=jnp.float32)
    h = jnp.maximum(acc.astype(_BF) + b1_ref[...].astype(_BF),
                    jnp.zeros((), _BF))
    return lax.dot_general(
        w2_ref[...].astype(_BF), h, (((1,), (1,)), ((), ())),
        preferred_element_type=jnp.float32).reshape(n)


def _node_body(x_ref, ei_ref,
               w1s_ref, b1s_ref, w2s_ref,
               w1d_ref, b1d_ref, w2d_ref,
               wsrc_ref, wdst_ref, srcp_ref, dstp_ref):
    i = pl.program_id(0)
    x_bf = x_ref[...].astype(_BF)
    wsrc_ref[...] = _score_head(x_bf, w1s_ref, b1s_ref, w2s_ref, NODE_BLK)
    wdst_ref[...] = _score_head(x_bf, w1d_ref, b1d_ref, w2d_ref, NODE_BLK)
    valid = i * EDGE_BLK + lax.iota(jnp.int32, EDGE_BLK) < N_EDGES
    zeros = jnp.zeros((EDGE_BLK,), jnp.int32)
    srcp_ref[...] = jnp.where(valid, ei_ref[0, :], zeros)
    dstp_ref[...] = jnp.where(valid, ei_ref[1, :], zeros)


def _edge_body(relt_ref, gum_ref, w1e_ref, b1e_ref, w2e_ref, b2_ref,
               wep_ref):
    i = pl.program_id(0)
    acc_e = lax.dot_general(
        relt_ref[...].astype(_BF), w1e_ref[...].astype(_BF),
        (((0,), (0,)), ((), ())), preferred_element_type=jnp.float32)
    h_e = jnp.maximum(acc_e.astype(_BF) + b1e_ref[...].astype(_BF),
                      jnp.zeros((), _BF))
    se = lax.dot_general(
        w2e_ref[...].astype(_BF), h_e, (((1,), (1,)), ((), ())),
        preferred_element_type=jnp.float32).reshape(EDGE_BLK)
    valid = i * EDGE_BLK + lax.iota(jnp.int32, EDGE_BLK) < N_EDGES
    wep_ref[...] = jnp.where(valid, se + b2_ref[0, 0] + gum_ref[...], -1e9)


def _gate_body(gs_ref, wep_ref, gate_ref, regsum_ref):
    i = pl.program_id(0)
    w = gs_ref[...] + wep_ref[...]
    g = 1.0 / (1.0 + jnp.exp(-2.0 * w))
    gate_ref[...] = g

    @pl.when(i == 0)
    def _init():
        regsum_ref[...] = jnp.zeros((1, 1), jnp.float32)

    regsum_ref[...] += jnp.sum(g).reshape(1, 1)


def _sc_gather_body(wsrc_hbm, wdst_hbm, src_hbm, dst_hbm, gs_hbm,
                    tbl_s, tbl_d, idx_s, idx_d, gs_v):
    wid = lax.axis_index("s") * 2 + lax.axis_index("c")
    base = pl.multiple_of(wid * CHUNK, 128)
    pltpu.sync_copy(wsrc_hbm, tbl_s)
    pltpu.sync_copy(wdst_hbm, tbl_d)
    pltpu.sync_copy(src_hbm.at[pl.ds(base, CHUNK)], idx_s)
    pltpu.sync_copy(dst_hbm.at[pl.ds(base, CHUNK)], idx_d)

    @plsc.parallel_loop(0, CHUNK, step=16, unroll=8)
    def _loop(o):
        vs = plsc.load_gather(tbl_s, [idx_s[pl.ds(o, 16)]])
        vd = plsc.load_gather(tbl_d, [idx_d[pl.ds(o, 16)]])
        gs_v[pl.ds(o, 16)] = vs + vd

    pltpu.sync_copy(gs_v, gs_hbm.at[pl.ds(base, CHUNK)])


def kernel(node_emb, edge_index, relation_emb,
           W1_src, b1_src, W2_src, b2_src,
           W1_dst, b1_dst, W2_dst, b2_dst,
           W1_edge, b1_edge, W2_edge, b2_edge):
    f32 = jnp.float32
    ei = edge_index.astype(jnp.int32)
    b2sum = (b2_src + b2_dst + b2_edge).reshape(1, 1)
    gum = jnp.asarray(_GUMBEL)

    full = lambda i: (0, 0)
    wsrc, wdst, src_p, dst_p = pl.pallas_call(
        _node_body,
        grid=(GRID,),
        in_specs=[
            pl.BlockSpec((NODE_BLK, NODE_DIM), lambda i: (i, 0)),
            pl.BlockSpec((2, EDGE_BLK), lambda i: (0, i)),
            pl.BlockSpec((NODE_DIM, HIDDEN), full),
            pl.BlockSpec((1, HIDDEN), full),
            pl.BlockSpec((1, HIDDEN), full),
            pl.BlockSpec((NODE_DIM, HIDDEN), full),
            pl.BlockSpec((1, HIDDEN), full),
            pl.BlockSpec((1, HIDDEN), full),
        ],
        out_specs=[
            pl.BlockSpec((NODE_BLK,), lambda i: (i,)),
            pl.BlockSpec((NODE_BLK,), lambda i: (i,)),
            pl.BlockSpec((EDGE_BLK,), lambda i: (i,)),
            pl.BlockSpec((EDGE_BLK,), lambda i: (i,)),
        ],
        out_shape=[
            jax.ShapeDtypeStruct((N_PAD,), f32),
            jax.ShapeDtypeStruct((N_PAD,), f32),
            jax.ShapeDtypeStruct((E_PAD,), jnp.int32),
            jax.ShapeDtypeStruct((E_PAD,), jnp.int32),
        ],
    )(node_emb, ei,
      W1_src, b1_src.reshape(1, HIDDEN), W2_src.reshape(1, HIDDEN),
      W1_dst, b1_dst.reshape(1, HIDDEN), W2_dst.reshape(1, HIDDEN))

    mesh = plsc.VectorSubcoreMesh(core_axis_name="c", subcore_axis_name="s")
    gs = pl.kernel(
        _sc_gather_body,
        out_type=jax.ShapeDtypeStruct((E_PAD,), f32),
        mesh=mesh,
        compiler_params=pltpu.CompilerParams(needs_layout_passes=False),
        scratch_types=[
            pltpu.VMEM((N_PAD,), f32),
            pltpu.VMEM((N_PAD,), f32),
            pltpu.VMEM((CHUNK,), jnp.int32),
            pltpu.VMEM((CHUNK,), jnp.int32),
            pltpu.VMEM((CHUNK,), f32),
        ],
    )(wsrc, wdst, src_p, dst_p)

    wep = pl.pallas_call(
        _edge_body,
        grid=(GRID,),
        in_specs=[
            pl.BlockSpec((EDGE_DIM, EDGE_BLK), lambda i: (0, i)),
            pl.BlockSpec((EDGE_BLK,), lambda i: (i,)),
            pl.BlockSpec((EDGE_DIM, HIDDEN), full),
            pl.BlockSpec((1, HIDDEN), full),
            pl.BlockSpec((1, HIDDEN), full),
            pl.BlockSpec((1, 1), full),
        ],
        out_specs=pl.BlockSpec((EDGE_BLK,), lambda i: (i,)),
        out_shape=jax.ShapeDtypeStruct((E_PAD,), f32),
    )(relation_emb.T, gum, W1_edge, b1_edge.reshape(1, HIDDEN),
      W2_edge.reshape(1, HIDDEN), b2sum)

    gate, regsum = pl.pallas_call(
        _gate_body,
        grid=(GRID,),
        in_specs=[
            pl.BlockSpec((EDGE_BLK,), lambda i: (i,)),
            pl.BlockSpec((EDGE_BLK,), lambda i: (i,)),
        ],
        out_specs=[
            pl.BlockSpec((EDGE_BLK,), lambda i: (i,)),
            pl.BlockSpec((1, 1), full),
        ],
        out_shape=[
            jax.ShapeDtypeStruct((N_EDGES,), f32),
            jax.ShapeDtypeStruct((1, 1), f32),
        ],
    )(gs, wep)

    aug_edge_weight = gate.reshape(N_EDGES, 1, 1)
    reg = 1.0 - regsum[0, 0] / np.float32(N_EDGES)
    return (reg, aug_edge_weight)


# speedup vs baseline: 18.7892x; 1.0294x over previous
import numpy as np
import jax
import jax.numpy as jnp
from jax import lax
from jax.experimental import pallas as pl
from jax.experimental.pallas import tpu as pltpu
from jax.experimental.pallas import tpu_sc as plsc

N_NODES = 10000
N_EDGES = 160000
NODE_DIM = 256
EDGE_DIM = 16
HIDDEN = 256

GRID = 10
N_PAD = 10240
NODE_BLK = 1024
EDGE_BLK = 16384
N_WORKERS = 32
E_PAD = 163840
CHUNK = E_PAD // N_WORKERS
LAST_VALID = N_EDGES - (N_WORKERS - 1) * CHUNK


def _gumbel_const() -> np.ndarray:
    def rotl(x, r):
        return ((x << np.uint32(r)) | (x >> np.uint32(32 - r))).astype(np.uint32)

    ks0, ks1 = np.uint32(0), np.uint32(123)
    ks2 = np.uint32(ks0 ^ ks1 ^ np.uint32(0x1BD11BDA))
    ks = [ks0, ks1, ks2]
    rot = [(13, 15, 26, 6), (17, 29, 16, 24)]
    x0 = np.full(N_EDGES, ks0, np.uint32)
    x1 = (np.arange(N_EDGES, dtype=np.uint32) + ks1).astype(np.uint32)
    for i in range(5):
        for r in rot[i % 2]:
            x0 = (x0 + x1).astype(np.uint32)
            x1 = rotl(x1, r)
            x1 = (x1 ^ x0).astype(np.uint32)
        x0 = (x0 + ks[(i + 1) % 3]).astype(np.uint32)
        x1 = (x1 + ks[(i + 2) % 3] + np.uint32(i + 1)).astype(np.uint32)
    bits = (x0 ^ x1).astype(np.uint32)
    u = ((bits >> np.uint32(9)) | np.uint32(0x3F800000)).view(np.float32) \
        - np.float32(1.0)
    bias = np.float32(0.0001)
    eps = (bias - (np.float32(1.0) - bias)) * u + (np.float32(1.0) - bias)
    g = np.log(eps, dtype=np.float32) - np.log(np.float32(1.0) - eps,
               dtype=np.float32)
    out = np.full(E_PAD, -1e9, np.float32)
    out[:N_EDGES] = g
    return out


_GUMBEL = _gumbel_const()
_BF = jnp.bfloat16


def _score_head(x_bf, w1_ref, b1_ref, w2_ref, n):
    acc = jnp.dot(x_bf, w1_ref[...].astype(_BF),
                  preferred_element_type=jnp.float32)
    h = jnp.maximum(acc.astype(_BF) + b1_ref[...].astype(_BF),
                    jnp.zeros((), _BF))
    return lax.dot_general(
        w2_ref[...].astype(_BF), h, (((1,), (1,)), ((), ())),
        preferred_element_type=jnp.float32).reshape(n)


def _tc_body(x_ref, relt_ref, ei_ref, gum_ref,
             w1s_ref, b1s_ref, w2s_ref,
             w1d_ref, b1d_ref, w2d_ref,
             w1e_ref, b1e_ref, w2e_ref, b2_ref,
             wsrc_ref, wdst_ref, wep_ref, srcp_ref, dstp_ref):
    i = pl.program_id(0)
    x_bf = x_ref[...].astype(_BF)
    wsrc_ref[...] = _score_head(x_bf, w1s_ref, b1s_ref, w2s_ref, NODE_BLK)
    wdst_ref[...] = _score_head(x_bf, w1d_ref, b1d_ref, w2d_ref, NODE_BLK)

    acc_e = lax.dot_general(
        relt_ref[...].astype(_BF), w1e_ref[...].astype(_BF),
        (((0,), (0,)), ((), ())), preferred_element_type=jnp.float32)
    h_e = jnp.maximum(acc_e.astype(_BF) + b1e_ref[...].astype(_BF),
                      jnp.zeros((), _BF))
    se = lax.dot_general(
        w2e_ref[...].astype(_BF), h_e, (((1,), (1,)), ((), ())),
        preferred_element_type=jnp.float32).reshape(EDGE_BLK)
    valid = i * EDGE_BLK + lax.iota(jnp.int32, EDGE_BLK) < N_EDGES
    wep_ref[...] = jnp.where(valid, se + b2_ref[0, 0] + gum_ref[...], -1e9)
    zeros = jnp.zeros((EDGE_BLK,), jnp.int32)
    srcp_ref[...] = jnp.where(valid, ei_ref[0, :], zeros)
    dstp_ref[...] = jnp.where(valid, ei_ref[1, :], zeros)


def _sc_gate_body(wsrc_hbm, wdst_hbm, src_hbm, dst_hbm, wep_hbm,
                  gate_hbm, part_hbm,
                  tbl_s, tbl_d, idx_s, idx_d, wep_v, gate_v, acc_v):
    wid = lax.axis_index("s") * 2 + lax.axis_index("c")
    base = pl.multiple_of(wid * CHUNK, 128)
    pltpu.sync_copy(wsrc_hbm, tbl_s)
    pltpu.sync_copy(wdst_hbm, tbl_d)
    pltpu.sync_copy(src_hbm.at[pl.ds(base, CHUNK)], idx_s)
    pltpu.sync_copy(dst_hbm.at[pl.ds(base, CHUNK)], idx_d)
    pltpu.sync_copy(wep_hbm.at[pl.ds(base, CHUNK)], wep_v)

    @plsc.parallel_loop(0, CHUNK, step=16, unroll=8,
                        carry=jnp.zeros((16,), jnp.float32))
    def acc(o, acc_in):
        vs = plsc.load_gather(tbl_s, [idx_s[pl.ds(o, 16)]])
        vd = plsc.load_gather(tbl_d, [idx_d[pl.ds(o, 16)]])
        w = wep_v[pl.ds(o, 16)] + vs + vd
        g = 1.0 / (1.0 + jnp.exp(-2.0 * w))
        gate_v[pl.ds(o, 16)] = g
        return acc_in + g

    acc_v[...] = acc

    @pl.when(wid < N_WORKERS - 1)
    def _copy_full():
        pltpu.sync_copy(gate_v, gate_hbm.at[pl.ds(base, CHUNK)])

    @pl.when(wid == N_WORKERS - 1)
    def _copy_last():
        pltpu.sync_copy(gate_v.at[pl.ds(0, LAST_VALID)],
                        gate_hbm.at[pl.ds(base, LAST_VALID)])

    pltpu.sync_copy(acc_v, part_hbm.at[wid])


def kernel(node_emb, edge_index, relation_emb,
           W1_src, b1_src, W2_src, b2_src,
           W1_dst, b1_dst, W2_dst, b2_dst,
           W1_edge, b1_edge, W2_edge, b2_edge):
    f32 = jnp.float32
    ei = edge_index.astype(jnp.int32)
    b2sum = (b2_src + b2_dst + b2_edge).reshape(1, 1)
    gum = jnp.asarray(_GUMBEL)

    full = lambda i: (0, 0)
    wsrc, wdst, wep, src_p, dst_p = pl.pallas_call(
        _tc_body,
        grid=(GRID,),
        in_specs=[
            pl.BlockSpec((NODE_BLK, NODE_DIM), lambda i: (i, 0)),
            pl.BlockSpec((EDGE_DIM, EDGE_BLK), lambda i: (0, i)),
            pl.BlockSpec((2, EDGE_BLK), lambda i: (0, i)),
            pl.BlockSpec((EDGE_BLK,), lambda i: (i,)),
            pl.BlockSpec((NODE_DIM, HIDDEN), full),
            pl.BlockSpec((1, HIDDEN), full),
            pl.BlockSpec((1, HIDDEN), full),
            pl.BlockSpec((NODE_DIM, HIDDEN), full),
            pl.BlockSpec((1, HIDDEN), full),
            pl.BlockSpec((1, HIDDEN), full),
            pl.BlockSpec((EDGE_DIM, HIDDEN), full),
            pl.BlockSpec((1, HIDDEN), full),
            pl.BlockSpec((1, HIDDEN), full),
            pl.BlockSpec((1, 1), full),
        ],
        out_specs=[
            pl.BlockSpec((NODE_BLK,), lambda i: (i,)),
            pl.BlockSpec((NODE_BLK,), lambda i: (i,)),
            pl.BlockSpec((EDGE_BLK,), lambda i: (i,)),
            pl.BlockSpec((EDGE_BLK,), lambda i: (i,)),
            pl.BlockSpec((EDGE_BLK,), lambda i: (i,)),
        ],
        out_shape=[
            jax.ShapeDtypeStruct((N_PAD,), f32),
            jax.ShapeDtypeStruct((N_PAD,), f32),
            jax.ShapeDtypeStruct((E_PAD,), f32),
            jax.ShapeDtypeStruct((E_PAD,), jnp.int32),
            jax.ShapeDtypeStruct((E_PAD,), jnp.int32),
        ],
    )(node_emb, relation_emb.T, ei, gum,
      W1_src, b1_src.reshape(1, HIDDEN), W2_src.reshape(1, HIDDEN),
      W1_dst, b1_dst.reshape(1, HIDDEN), W2_dst.reshape(1, HIDDEN),
      W1_edge, b1_edge.reshape(1, HIDDEN), W2_edge.reshape(1, HIDDEN),
      b2sum)

    mesh = plsc.VectorSubcoreMesh(core_axis_name="c", subcore_axis_name="s")
    sc_gate = pl.kernel(
        _sc_gate_body,
        out_type=[
            jax.ShapeDtypeStruct((N_EDGES,), f32),
            jax.ShapeDtypeStruct((N_WORKERS, 16), f32),
        ],
        mesh=mesh,
        compiler_params=pltpu.CompilerParams(needs_layout_passes=False),
        scratch_types=[
            pltpu.VMEM((N_PAD,), f32),
            pltpu.VMEM((N_PAD,), f32),
            pltpu.VMEM((CHUNK,), jnp.int32),
            pltpu.VMEM((CHUNK,), jnp.int32),
            pltpu.VMEM((CHUNK,), f32),
            pltpu.VMEM((CHUNK,), f32),
            pltpu.VMEM((16,), f32),
        ],
    )
    gate, parts = sc_gate(wsrc, wdst, src_p, dst_p, wep)

    aug_edge_weight = gate.reshape(N_EDGES, 1, 1)
    reg = 1.0 - jnp.sum(parts) / np.float32(N_EDGES)
    return (reg, aug_edge_weight)
